# baseline (speedup 1.0000x reference)
import functools

import jax
import jax.numpy as jnp
from jax import lax
from jax.experimental import pallas as pl
from jax.experimental.pallas import tpu as pltpu
from jax.experimental.pallas import tpu_sc as plsc

B, L, V, D, H = 4096, 200, 1000000, 64, 64
N = B * L

NC, NS = 2, 16
NW = NC * NS
ROWS_PER_W = N // NW
CHUNK = 128
NCHUNK = ROWS_PER_W // CHUNK

ROW_BLK = 8000


def _proj_body(emb_ref, w_ref, out_ref):
    out_ref[...] = lax.dot_general(
        emb_ref[...], w_ref[...],
        dimension_numbers=(((1,), (1,)), ((), ())),
        preferred_element_type=jnp.float32,
    )


def _project(emb, W):
    return pl.pallas_call(
        _proj_body,
        grid=(V // ROW_BLK,),
        in_specs=[
            pl.BlockSpec((ROW_BLK, D), lambda i: (i, 0)),
            pl.BlockSpec((H, D), lambda i: (0, 0)),
        ],
        out_specs=pl.BlockSpec((ROW_BLK, H), lambda i: (i, 0)),
        out_shape=jax.ShapeDtypeStruct((V, H), jnp.float32),
    )(emb, W)


_mesh = plsc.VectorSubcoreMesh(core_axis_name="c", subcore_axis_name="s")


@functools.partial(
    pl.kernel,
    mesh=_mesh,
    compiler_params=pltpu.CompilerParams(use_tc_tiling_on_sc=False),
    out_type=[
        jax.ShapeDtypeStruct((N, H), jnp.float32),
        jax.ShapeDtypeStruct((N, H), jnp.float32),
    ],
    scratch_types=[
        pltpu.VMEM((NCHUNK, CHUNK), jnp.int32),
        pltpu.VMEM((CHUNK, H), jnp.float32),
        pltpu.SemaphoreType.DMA,
    ],
)
def _gather(proj_hbm, idx1_hbm, idx2_hbm, out1_hbm, out2_hbm,
            idx_v, rows_v, sem):
    wid = lax.axis_index("s") * NC + lax.axis_index("c")
    base = wid * ROWS_PER_W
    for idx_hbm, out_hbm in ((idx1_hbm, out1_hbm), (idx2_hbm, out2_hbm)):
        pltpu.sync_copy(idx_hbm.at[wid], idx_v)

        def body(j, carry, out_hbm=out_hbm):
            pltpu.async_copy(proj_hbm.at[idx_v.at[j]], rows_v, sem).wait()
            pltpu.sync_copy(rows_v, out_hbm.at[pl.ds(base + j * CHUNK, CHUNK)])
            return carry

        lax.fori_loop(0, NCHUNK, body, 0)


def kernel(sent1, sent2, emb, W):
    proj = _project(emb, W)
    idx1 = sent1.reshape(NW, NCHUNK, CHUNK).astype(jnp.int32)
    idx2 = sent2.reshape(NW, NCHUNK, CHUNK).astype(jnp.int32)
    out1, out2 = _gather(proj, idx1, idx2)
    return (out1.reshape(B, L, H), out2.reshape(B, L, H))


# speedup vs baseline: 1.4394x; 1.4394x over previous
import functools

import jax
import jax.numpy as jnp
from jax import lax
from jax.experimental import pallas as pl
from jax.experimental.pallas import tpu as pltpu
from jax.experimental.pallas import tpu_sc as plsc

B, L, V, D, H = 4096, 200, 1000000, 64, 64
N = B * L

NC, NS = 2, 16
NW = NC * NS
ROWS_PER_W = N // NW
CHUNK = 128
NCHUNK = ROWS_PER_W // CHUNK

ROW_BLK = 8192
NBLK = (V + ROW_BLK - 1) // ROW_BLK
VP = NBLK * ROW_BLK
HALF = ROW_BLK // 2


def _proj_body(embt_ref, w_ref, out_ref):
    y = lax.dot_general(
        embt_ref[...], w_ref[...],
        dimension_numbers=(((0,), (1,)), ((), ())),
        preferred_element_type=jnp.float32,
    )
    out_ref[:, :H] = y[:HALF]
    out_ref[:, H:] = y[HALF:]


def _project(embt, W):
    return pl.pallas_call(
        _proj_body,
        grid=(NBLK,),
        in_specs=[
            pl.BlockSpec((D, ROW_BLK), lambda i: (0, i)),
            pl.BlockSpec((H, D), lambda i: (0, 0)),
        ],
        out_specs=pl.BlockSpec((HALF, 2 * H), lambda i: (i, 0)),
        out_shape=jax.ShapeDtypeStruct((VP // 2, 2 * H), jnp.float32),
    )(embt, W)


_mesh = plsc.VectorSubcoreMesh(core_axis_name="c", subcore_axis_name="s")


@functools.partial(
    pl.kernel,
    mesh=_mesh,
    compiler_params=pltpu.CompilerParams(use_tc_tiling_on_sc=False),
    out_type=[
        jax.ShapeDtypeStruct((N, H), jnp.float32),
        jax.ShapeDtypeStruct((N, H), jnp.float32),
    ],
    scratch_types=[
        pltpu.VMEM((NCHUNK, CHUNK), jnp.int32),
        pltpu.VMEM((CHUNK, H), jnp.float32),
        pltpu.SemaphoreType.DMA,
    ],
)
def _gather(proj_hbm, idx1_hbm, idx2_hbm, out1_hbm, out2_hbm,
            idx_v, rows_v, sem):
    wid = lax.axis_index("s") * NC + lax.axis_index("c")
    base = wid * ROWS_PER_W
    for idx_hbm, out_hbm in ((idx1_hbm, out1_hbm), (idx2_hbm, out2_hbm)):
        pltpu.sync_copy(idx_hbm.at[wid], idx_v)

        def body(j, carry, out_hbm=out_hbm):
            pltpu.async_copy(proj_hbm.at[idx_v.at[j]], rows_v, sem).wait()
            pltpu.sync_copy(rows_v, out_hbm.at[pl.ds(base + j * CHUNK, CHUNK)])
            return carry

        lax.fori_loop(0, NCHUNK, body, 0)


def _sigma(v):
    v = v.astype(jnp.int32)
    return ((v >> 13) << 13) + ((v & (HALF - 1)) << 1) + ((v >> 12) & 1)


def kernel(sent1, sent2, emb, W):
    proj = _project(emb.T, W).reshape(VP, H)
    idx1 = _sigma(sent1).reshape(NW, NCHUNK, CHUNK)
    idx2 = _sigma(sent2).reshape(NW, NCHUNK, CHUNK)
    out1, out2 = _gather(proj, idx1, idx2)
    return (out1.reshape(B, L, H), out2.reshape(B, L, H))


# speedup vs baseline: 1.7209x; 1.1955x over previous
import functools

import jax
import jax.numpy as jnp
from jax import lax
from jax.experimental import pallas as pl
from jax.experimental.pallas import tpu as pltpu
from jax.experimental.pallas import tpu_sc as plsc

B, L, V, D, H = 4096, 200, 1000000, 64, 64
N = B * L

NC, NS = 2, 16
NW = NC * NS
ROWS_PER_W = N // NW
CHUNK = 128
NCHUNK = ROWS_PER_W // CHUNK

ROW_BLK = 8192
NBLK = (V + ROW_BLK - 1) // ROW_BLK
VP = NBLK * ROW_BLK
HALF = ROW_BLK // 2


def _proj_body(embt_ref, w_ref, out_ref):
    y = lax.dot_general(
        embt_ref[...], w_ref[...],
        dimension_numbers=(((0,), (1,)), ((), ())),
        preferred_element_type=jnp.float32,
    )
    out_ref[:, :H] = y[:HALF]
    out_ref[:, H:] = y[HALF:]


def _project(embt, W):
    return pl.pallas_call(
        _proj_body,
        grid=(NBLK,),
        in_specs=[
            pl.BlockSpec((D, ROW_BLK), lambda i: (0, i)),
            pl.BlockSpec((H, D), lambda i: (0, 0)),
        ],
        out_specs=pl.BlockSpec((HALF, 2 * H), lambda i: (i, 0)),
        out_shape=jax.ShapeDtypeStruct((VP // 2, 2 * H), jnp.float32),
    )(embt, W)


_mesh = plsc.VectorSubcoreMesh(core_axis_name="c", subcore_axis_name="s")


NBUF = 8
DEPTH = NBUF // 2
NLAP = NCHUNK // NBUF


@functools.partial(
    pl.kernel,
    mesh=_mesh,
    compiler_params=pltpu.CompilerParams(use_tc_tiling_on_sc=False),
    out_type=[
        jax.ShapeDtypeStruct((N, H), jnp.float32),
        jax.ShapeDtypeStruct((N, H), jnp.float32),
    ],
    scratch_types=[
        pltpu.VMEM((NCHUNK, CHUNK), jnp.int32),
        pltpu.VMEM((NBUF, CHUNK, H), jnp.float32),
        [pltpu.SemaphoreType.DMA] * NBUF,
        [pltpu.SemaphoreType.DMA] * NBUF,
    ],
)
def _gather(proj_hbm, idx1_hbm, idx2_hbm, out1_hbm, out2_hbm,
            idx_v, rows_v, gsem, ssem):
    wid = lax.axis_index("s") * NC + lax.axis_index("c")
    base = wid * ROWS_PER_W

    for idx_hbm, out_hbm in ((idx1_hbm, out1_hbm), (idx2_hbm, out2_hbm)):
        pltpu.sync_copy(idx_hbm.at[wid], idx_v)

        def g_copy(j, b, out_hbm=out_hbm):
            return pltpu.make_async_copy(
                proj_hbm.at[idx_v.at[j]], rows_v.at[b], gsem[b])

        def s_copy(j, b, out_hbm=out_hbm):
            return pltpu.make_async_copy(
                rows_v.at[b], out_hbm.at[pl.ds(base + j * CHUNK, CHUNK)],
                ssem[b])

        def emit(j, b, issue_next, wait_next_store):
            g_copy(j, b).wait()
            s_copy(j, b).start()
            if issue_next:
                nxt = j + DEPTH
                b2 = (b + DEPTH) % NBUF
                if wait_next_store:
                    s_copy(nxt - NBUF, b2).wait()
                g_copy(nxt, b2).start()

        for b in range(DEPTH):
            g_copy(b, b).start()
        for b in range(NBUF):
            emit(b, b, True, b >= DEPTH)

        def lap(t, carry):
            j0 = t * NBUF
            for b in range(NBUF):
                emit(j0 + b, b, True, True)
            return carry

        lax.fori_loop(1, NLAP - 1, lap, 0)

        j0 = (NLAP - 1) * NBUF
        for b in range(NBUF):
            emit(j0 + b, b, b < DEPTH, True)
        for b in range(NBUF):
            s_copy(j0 + b, b).wait()


def _sigma(v):
    v = v.astype(jnp.int32)
    return ((v >> 13) << 13) + ((v & (HALF - 1)) << 1) + ((v >> 12) & 1)


def kernel(sent1, sent2, emb, W):
    proj = _project(emb.T, W).reshape(VP, H)
    idx1 = _sigma(sent1).reshape(NW, NCHUNK, CHUNK)
    idx2 = _sigma(sent2).reshape(NW, NCHUNK, CHUNK)
    out1, out2 = _gather(proj, idx1, idx2)
    return (out1.reshape(B, L, H), out2.reshape(B, L, H))
